# baseline (speedup 1.0000x reference)
import jax
import jax.numpy as jnp
from jax import lax
from jax.experimental import pallas as pl
from jax.experimental.pallas import tpu as pltpu
from jax.experimental.pallas import tpu_sc as plsc

N_NODES = 10000
N_EDGES = 320000
D_IN = 128
D_OUT = 128
SLOPE = 0.2

NC = 2
NS = 16
NW = NC * NS
F_PER_TILE = D_OUT // NW
E_PER_W = N_EDGES // NW
CHUNK = 10000
CHUNK3 = 3200
N_PAD = 10240
SLICE = N_PAD // NS
MASK16 = 0xFFFF


def _split_sd(sd):
    src16 = jnp.bitwise_and(sd, MASK16)
    dst16 = lax.shift_right_logical(sd, 16)
    return src16, dst16


def _tc_body(x_ref, w_ref, att8_ref, ht_ref, a8_ref):
    xb = x_ref[...]
    ht = lax.dot_general(w_ref[...], xb, (((0,), (1,)), ((), ())),
                         preferred_element_type=jnp.float32)
    ht_ref[...] = ht
    a8_ref[...] = lax.dot_general(att8_ref[...], ht, (((1,), (0,)), ((), ())),
                                  preferred_element_type=jnp.float32)


def _k1(x, W, att8):
    bn = 1280
    grid = N_PAD // bn
    return pl.pallas_call(
        _tc_body,
        grid=(grid,),
        in_specs=[
            pl.BlockSpec((bn, D_IN), lambda i: (i, 0)),
            pl.BlockSpec((D_IN, D_OUT), lambda i: (0, 0)),
            pl.BlockSpec((8, D_IN), lambda i: (0, 0)),
        ],
        out_specs=[
            pl.BlockSpec((D_OUT, bn), lambda i: (0, i)),
            pl.BlockSpec((8, bn), lambda i: (0, i)),
        ],
        out_shape=[
            jax.ShapeDtypeStruct((D_OUT, N_PAD), jnp.float32),
            jax.ShapeDtypeStruct((8, N_PAD), jnp.float32),
        ],
    )(x, W, att8)


def _k2_body(sd_hbm, a8_hbm, w_hbm, dpart_hbm,
             asrc_v, adst_v, den_v, sd_v, wv_v, red_v, out_v, shared):
    c = lax.axis_index("c")
    s = lax.axis_index("s")
    wid = c * NS + s
    pltpu.sync_copy(a8_hbm.at[0], asrc_v)
    pltpu.sync_copy(a8_hbm.at[1], adst_v)

    @plsc.parallel_loop(0, N_PAD // 16, unroll=4)
    def zero_body(j):
        den_v[pl.ds(j * 16, 16)] = jnp.zeros((16,), jnp.float32)

    ebase = wid * E_PER_W

    def chunk_body(k, _):
        base = ebase + k * CHUNK
        pltpu.sync_copy(sd_hbm.at[pl.ds(base, CHUNK)], sd_v)

        @plsc.parallel_loop(0, CHUNK // 16, unroll=8)
        def grp_body(g):
            src16, dst16 = _split_sd(sd_v[pl.ds(g * 16, 16)])
            al = (plsc.load_gather(asrc_v, [src16])
                  + plsc.load_gather(adst_v, [dst16]))
            al = jnp.where(al >= 0.0, al, al * SLOPE)
            w16 = jnp.exp(al)
            wv_v[pl.ds(g * 16, 16)] = w16
            plsc.addupdate_scatter(den_v, [dst16], w16)
        pltpu.sync_copy(wv_v, w_hbm.at[pl.ds(base, CHUNK)])
        return 0
    lax.fori_loop(0, E_PER_W // CHUNK, chunk_body, 0)

    pltpu.sync_copy(den_v, shared.at[s])
    plsc.subcore_barrier()
    pltpu.sync_copy(shared.at[:, pl.ds(s * SLICE, SLICE)], red_v)

    @plsc.parallel_loop(0, SLICE // 16, unroll=2)
    def red_body(j):
        acc = jnp.zeros((16,), jnp.float32)
        for r in range(NS):
            acc = acc + red_v[r, pl.ds(j * 16, 16)]
        out_v[pl.ds(j * 16, 16)] = acc
    pltpu.sync_copy(out_v, dpart_hbm.at[c, pl.ds(s * SLICE, SLICE)])


def _k2(srcdst, a8):
    mesh = plsc.VectorSubcoreMesh(core_axis_name="c", subcore_axis_name="s")
    f = pl.kernel(
        _k2_body,
        out_type=[
            jax.ShapeDtypeStruct((N_EDGES,), jnp.float32),
            jax.ShapeDtypeStruct((NC, N_PAD), jnp.float32),
        ],
        mesh=mesh,
        compiler_params=pltpu.CompilerParams(needs_layout_passes=False),
        scratch_types=[
            pltpu.VMEM((N_PAD,), jnp.float32),
            pltpu.VMEM((N_PAD,), jnp.float32),
            pltpu.VMEM((N_PAD,), jnp.float32),
            pltpu.VMEM((CHUNK,), jnp.int32),
            pltpu.VMEM((CHUNK,), jnp.float32),
            pltpu.VMEM((NS, SLICE), jnp.float32),
            pltpu.VMEM((SLICE,), jnp.float32),
            pltpu.VMEM_SHARED((NS, N_PAD), jnp.float32),
        ],
    )
    return f(srcdst, a8)


def _k3_body(ht_hbm, sd_hbm, w_hbm, dpart_hbm, bias_hbm, out_hbm,
             hstage_v, hp_v, acc_v, sd_v, wv_v, den_v, den2_v, bias_v, sems):
    c = lax.axis_index("c")
    s = lax.axis_index("s")
    t = c * NS + s
    fbase = t * F_PER_TILE
    nch = N_EDGES // CHUNK3

    def start_chunk(k, slot):
        base = k * CHUNK3
        pltpu.async_copy(sd_hbm.at[pl.ds(base, CHUNK3)], sd_v.at[slot],
                         sems.at[slot])
        pltpu.async_copy(w_hbm.at[pl.ds(base, CHUNK3)], wv_v.at[slot],
                         sems.at[slot])

    def wait_chunk(slot):
        pltpu.make_async_copy(sd_hbm.at[pl.ds(0, CHUNK3)], sd_v.at[slot],
                              sems.at[slot]).wait()
        pltpu.make_async_copy(w_hbm.at[pl.ds(0, CHUNK3)], wv_v.at[slot],
                              sems.at[slot]).wait()

    start_chunk(0, 0)

    for p in range(F_PER_TILE // 2):
        pltpu.sync_copy(ht_hbm.at[pl.ds(fbase + 2 * p, 2)], hstage_v)

        @plsc.parallel_loop(0, N_PAD // 16, unroll=4)
        def pack_body(j, p=p):
            a = hstage_v[0, pl.ds(j * 16, 16)]
            b = hstage_v[1, pl.ds(j * 16, 16)]
            packed = plsc.pack(a, b, format=plsc.PackFormat.INTERLEAVED)
            hp_v[p, pl.ds(j * 16, 16)] = plsc.bitcast(packed, jnp.int32)

    @plsc.parallel_loop(0, N_PAD // 16, unroll=4)
    def zero_body(j):
        for f in range(F_PER_TILE):
            acc_v[f, pl.ds(j * 16, 16)] = jnp.zeros((16,), jnp.float32)

    def chunk_body(k, _):
        slot = lax.rem(k, 2)

        @pl.when(k + 1 < nch)
        def _():
            start_chunk(k + 1, 1 - slot)
        wait_chunk(slot)

        @plsc.parallel_loop(0, CHUNK3 // 16, unroll=12)
        def grp_body(g):
            src16, dst16 = _split_sd(sd_v[slot, pl.ds(g * 16, 16)])
            w16 = wv_v[slot, pl.ds(g * 16, 16)]
            for p in range(F_PER_TILE // 2):
                pidx = jnp.full((16,), p, jnp.int32)
                words = plsc.load_gather(hp_v, [pidx, src16])
                rows = plsc.unpack(plsc.bitcast(words, jnp.bfloat16),
                                   format=plsc.PackFormat.INTERLEAVED)
                for q in range(2):
                    fidx = jnp.full((16,), 2 * p + q, jnp.int32)
                    plsc.addupdate_scatter(acc_v, [fidx, dst16],
                                           rows[q] * w16)
        return 0
    lax.fori_loop(0, nch, chunk_body, 0)

    pltpu.sync_copy(dpart_hbm.at[0], den_v)
    pltpu.sync_copy(dpart_hbm.at[1], den2_v)
    pltpu.sync_copy(bias_hbm, bias_v)

    bfs = [plsc.load_gather(bias_v, [jnp.full((16,), 1, jnp.int32) * (fbase + f)])
           for f in range(F_PER_TILE)]

    @plsc.parallel_loop(0, N_PAD // 16, unroll=2)
    def norm_body(j):
        dsum = den_v[pl.ds(j * 16, 16)] + den2_v[pl.ds(j * 16, 16)] + 1e-16
        rden = 1.0 / dsum
        for f in range(F_PER_TILE):
            v = acc_v[f, pl.ds(j * 16, 16)] * rden + bfs[f]
            v = jnp.where(v > 0.0, v, jnp.exp(v) - 1.0)
            acc_v[f, pl.ds(j * 16, 16)] = v
    pltpu.sync_copy(acc_v, out_hbm.at[pl.ds(fbase, F_PER_TILE)])


def _k3(ht, srcdst, w_e, dparts, bias):
    mesh = plsc.VectorSubcoreMesh(core_axis_name="c", subcore_axis_name="s")
    f = pl.kernel(
        _k3_body,
        out_type=jax.ShapeDtypeStruct((D_OUT, N_PAD), jnp.float32),
        mesh=mesh,
        compiler_params=pltpu.CompilerParams(needs_layout_passes=False),
        scratch_types=[
            pltpu.VMEM((2, N_PAD), jnp.float32),
            pltpu.VMEM((F_PER_TILE // 2, N_PAD), jnp.int32),
            pltpu.VMEM((F_PER_TILE, N_PAD), jnp.float32),
            pltpu.VMEM((2, CHUNK3), jnp.int32),
            pltpu.VMEM((2, CHUNK3), jnp.float32),
            pltpu.VMEM((N_PAD,), jnp.float32),
            pltpu.VMEM((N_PAD,), jnp.float32),
            pltpu.VMEM((D_OUT,), jnp.float32),
            pltpu.SemaphoreType.DMA((2,)),
        ],
    )
    return f(ht, srcdst, w_e, dparts, bias)


def kernel(x, edge_index, W, att_src, att_dst, bias):
    att8 = jnp.concatenate(
        [att_src, att_dst, jnp.zeros((6, D_IN), jnp.float32)], axis=0)
    xp = jnp.pad(x, ((0, N_PAD - N_NODES), (0, 0)))
    srcdst = jnp.bitwise_or(edge_index[0],
                            lax.shift_left(edge_index[1], 16))
    ht, a8 = _k1(xp, W, att8)
    w_e, dparts = _k2(srcdst, a8)
    out_t = _k3(ht, srcdst, w_e, dparts, bias)
    return out_t[:, :N_NODES].T


# speedup vs baseline: 1.1435x; 1.1435x over previous
import jax
import jax.numpy as jnp
from jax import lax
from jax.experimental import pallas as pl
from jax.experimental.pallas import tpu as pltpu
from jax.experimental.pallas import tpu_sc as plsc

N_NODES = 10000
N_EDGES = 320000
D_IN = 128
D_OUT = 128
SLOPE = 0.2

NC = 2
NS = 16
NW = NC * NS
F_PER_TILE = D_OUT // NW
E_PER_W = N_EDGES // NW
CHUNK = 10000
CHUNK3 = 3200
N_PAD = 10240
SLICE = N_PAD // NS
MASK16 = 0xFFFF


def _split_sd(sd):
    src16 = jnp.bitwise_and(sd, MASK16)
    dst16 = lax.shift_right_logical(sd, 16)
    return src16, dst16


def _tc_body(x_ref, w_ref, att8_ref, ht_ref, a8_ref):
    xb = x_ref[...]
    ht = lax.dot_general(w_ref[...], xb, (((0,), (1,)), ((), ())),
                         preferred_element_type=jnp.float32)
    ht_ref[...] = ht
    a8_ref[...] = lax.dot_general(att8_ref[...], ht, (((1,), (0,)), ((), ())),
                                  preferred_element_type=jnp.float32)


def _k1(x, W, att8):
    bn = 1280
    grid = N_PAD // bn
    return pl.pallas_call(
        _tc_body,
        grid=(grid,),
        in_specs=[
            pl.BlockSpec((bn, D_IN), lambda i: (i, 0)),
            pl.BlockSpec((D_IN, D_OUT), lambda i: (0, 0)),
            pl.BlockSpec((8, D_IN), lambda i: (0, 0)),
        ],
        out_specs=[
            pl.BlockSpec((D_OUT, bn), lambda i: (0, i)),
            pl.BlockSpec((8, bn), lambda i: (0, i)),
        ],
        out_shape=[
            jax.ShapeDtypeStruct((D_OUT, N_PAD), jnp.float32),
            jax.ShapeDtypeStruct((8, N_PAD), jnp.float32),
        ],
    )(x, W, att8)


def _k2_body(sd_hbm, a8_hbm, w_hbm, dpart_hbm,
             asrc_v, adst_v, den_v, sd_v, wv_v, red_v, out_v, shared):
    c = lax.axis_index("c")
    s = lax.axis_index("s")
    wid = c * NS + s
    pltpu.sync_copy(a8_hbm.at[0], asrc_v)
    pltpu.sync_copy(a8_hbm.at[1], adst_v)

    @plsc.parallel_loop(0, N_PAD // 16, unroll=4)
    def zero_body(j):
        den_v[pl.ds(j * 16, 16)] = jnp.zeros((16,), jnp.float32)

    ebase = wid * E_PER_W

    def chunk_body(k, _):
        base = ebase + k * CHUNK
        pltpu.sync_copy(sd_hbm.at[pl.ds(base, CHUNK)], sd_v)

        @plsc.parallel_loop(0, CHUNK // 16, unroll=8)
        def grp_body(g):
            src16, dst16 = _split_sd(sd_v[pl.ds(g * 16, 16)])
            al = (plsc.load_gather(asrc_v, [src16])
                  + plsc.load_gather(adst_v, [dst16]))
            al = jnp.where(al >= 0.0, al, al * SLOPE)
            w16 = jnp.exp(al)
            wv_v[pl.ds(g * 16, 16)] = w16
            plsc.addupdate_scatter(den_v, [dst16], w16)
        pltpu.sync_copy(wv_v, w_hbm.at[pl.ds(base, CHUNK)])
        return 0
    lax.fori_loop(0, E_PER_W // CHUNK, chunk_body, 0)

    pltpu.sync_copy(den_v, shared.at[s])
    plsc.subcore_barrier()
    pltpu.sync_copy(shared.at[:, pl.ds(s * SLICE, SLICE)], red_v)

    @plsc.parallel_loop(0, SLICE // 16, unroll=2)
    def red_body(j):
        acc = jnp.zeros((16,), jnp.float32)
        for r in range(NS):
            acc = acc + red_v[r, pl.ds(j * 16, 16)]
        out_v[pl.ds(j * 16, 16)] = acc
    pltpu.sync_copy(out_v, dpart_hbm.at[c, pl.ds(s * SLICE, SLICE)])


def _k2(srcdst, a8):
    mesh = plsc.VectorSubcoreMesh(core_axis_name="c", subcore_axis_name="s")
    f = pl.kernel(
        _k2_body,
        out_type=[
            jax.ShapeDtypeStruct((N_EDGES,), jnp.float32),
            jax.ShapeDtypeStruct((NC, N_PAD), jnp.float32),
        ],
        mesh=mesh,
        compiler_params=pltpu.CompilerParams(needs_layout_passes=False),
        scratch_types=[
            pltpu.VMEM((N_PAD,), jnp.float32),
            pltpu.VMEM((N_PAD,), jnp.float32),
            pltpu.VMEM((N_PAD,), jnp.float32),
            pltpu.VMEM((CHUNK,), jnp.int32),
            pltpu.VMEM((CHUNK,), jnp.float32),
            pltpu.VMEM((NS, SLICE), jnp.float32),
            pltpu.VMEM((SLICE,), jnp.float32),
            pltpu.VMEM_SHARED((NS, N_PAD), jnp.float32),
        ],
    )
    return f(srcdst, a8)


def _k3_body(ht_hbm, sd_hbm, w_hbm, dpart_hbm, bias_hbm, out_hbm,
             hstage_v, hp_v, acc_v, sd_v, wv_v, den_v, den2_v, bias_v, sems):
    c = lax.axis_index("c")
    s = lax.axis_index("s")
    t = c * NS + s
    fbase = t * F_PER_TILE
    nch = N_EDGES // CHUNK3

    def start_chunk(k, slot):
        base = k * CHUNK3
        pltpu.async_copy(sd_hbm.at[pl.ds(base, CHUNK3)], sd_v.at[slot],
                         sems.at[slot])
        pltpu.async_copy(w_hbm.at[pl.ds(base, CHUNK3)], wv_v.at[slot],
                         sems.at[slot])

    def wait_chunk(slot):
        pltpu.make_async_copy(sd_hbm.at[pl.ds(0, CHUNK3)], sd_v.at[slot],
                              sems.at[slot]).wait()
        pltpu.make_async_copy(w_hbm.at[pl.ds(0, CHUNK3)], wv_v.at[slot],
                              sems.at[slot]).wait()

    start_chunk(0, 0)

    for p in range(F_PER_TILE // 2):
        pltpu.sync_copy(ht_hbm.at[pl.ds(fbase + 2 * p, 2)], hstage_v)

        @plsc.parallel_loop(0, N_PAD // 16, unroll=4)
        def pack_body(j, p=p):
            a = hstage_v[0, pl.ds(j * 16, 16)]
            b = hstage_v[1, pl.ds(j * 16, 16)]
            packed = plsc.pack(a, b, format=plsc.PackFormat.INTERLEAVED)
            hp_v[p, pl.ds(j * 16, 16)] = plsc.bitcast(packed, jnp.int32)

    @plsc.parallel_loop(0, N_PAD // 16, unroll=4)
    def zero_body(j):
        for f in range(F_PER_TILE):
            acc_v[f, pl.ds(j * 16, 16)] = jnp.zeros((16,), jnp.float32)

    def chunk_body(k, _):
        slot = lax.rem(k, 2)

        @pl.when(k + 1 < nch)
        def _():
            start_chunk(k + 1, 1 - slot)
        wait_chunk(slot)

        @plsc.parallel_loop(0, CHUNK3 // 16, unroll=8)
        def grp_body(g):
            src16, dst16 = _split_sd(sd_v[slot, pl.ds(g * 16, 16)])
            w16 = wv_v[slot, pl.ds(g * 16, 16)]
            for p in range(F_PER_TILE // 2):
                pidx = jnp.full((16,), p, jnp.int32)
                words = plsc.load_gather(hp_v, [pidx, src16])
                rows = plsc.unpack(plsc.bitcast(words, jnp.bfloat16),
                                   format=plsc.PackFormat.INTERLEAVED)
                for q in range(2):
                    fidx = jnp.full((16,), 2 * p + q, jnp.int32)
                    plsc.addupdate_scatter(acc_v, [fidx, dst16],
                                           rows[q] * w16)
        return 0
    lax.fori_loop(0, nch, chunk_body, 0)

    pltpu.sync_copy(dpart_hbm.at[0], den_v)
    pltpu.sync_copy(dpart_hbm.at[1], den2_v)
    pltpu.sync_copy(bias_hbm, bias_v)

    bfs = [plsc.load_gather(bias_v, [jnp.full((16,), 1, jnp.int32) * (fbase + f)])
           for f in range(F_PER_TILE)]

    @plsc.parallel_loop(0, N_PAD // 16, unroll=2)
    def norm_body(j):
        dsum = den_v[pl.ds(j * 16, 16)] + den2_v[pl.ds(j * 16, 16)] + 1e-16
        rden = 1.0 / dsum
        for f in range(F_PER_TILE):
            v = acc_v[f, pl.ds(j * 16, 16)] * rden + bfs[f]
            v = jnp.where(v > 0.0, v, jnp.exp(v) - 1.0)
            acc_v[f, pl.ds(j * 16, 16)] = v
    pltpu.sync_copy(acc_v, out_hbm.at[pl.ds(fbase, F_PER_TILE)])


def _k3(ht, srcdst, w_e, dparts, bias):
    mesh = plsc.VectorSubcoreMesh(core_axis_name="c", subcore_axis_name="s")
    f = pl.kernel(
        _k3_body,
        out_type=jax.ShapeDtypeStruct((D_OUT, N_PAD), jnp.float32),
        mesh=mesh,
        compiler_params=pltpu.CompilerParams(needs_layout_passes=False),
        scratch_types=[
            pltpu.VMEM((2, N_PAD), jnp.float32),
            pltpu.VMEM((F_PER_TILE // 2, N_PAD), jnp.int32),
            pltpu.VMEM((F_PER_TILE, N_PAD), jnp.float32),
            pltpu.VMEM((2, CHUNK3), jnp.int32),
            pltpu.VMEM((2, CHUNK3), jnp.float32),
            pltpu.VMEM((N_PAD,), jnp.float32),
            pltpu.VMEM((N_PAD,), jnp.float32),
            pltpu.VMEM((D_OUT,), jnp.float32),
            pltpu.SemaphoreType.DMA((2,)),
        ],
    )
    return f(ht, srcdst, w_e, dparts, bias)


def kernel(x, edge_index, W, att_src, att_dst, bias):
    att8 = jnp.concatenate(
        [att_src, att_dst, jnp.zeros((6, D_IN), jnp.float32)], axis=0)
    xp = jnp.pad(x, ((0, N_PAD - N_NODES), (0, 0)))
    srcdst = jnp.bitwise_or(edge_index[0],
                            lax.shift_left(edge_index[1], 16))
    ht, a8 = _k1(xp, W, att8)
    w_e, dparts = _k2(srcdst, a8)
    out_t = _k3(ht, srcdst, w_e, dparts, bias)
    return out_t[:, :N_NODES].T


# speedup vs baseline: 1.1456x; 1.0018x over previous
import jax
import jax.numpy as jnp
from jax import lax
from jax.experimental import pallas as pl
from jax.experimental.pallas import tpu as pltpu
from jax.experimental.pallas import tpu_sc as plsc

N_NODES = 10000
N_EDGES = 320000
D_IN = 128
D_OUT = 128
SLOPE = 0.2

NC = 2
NS = 16
NW = NC * NS
F_PER_TILE = D_OUT // NW
E_PER_W = N_EDGES // NW
CHUNK = 10000
CHUNK3 = 3200
N_PAD = 10240
SLICE = N_PAD // NS
MASK16 = 0xFFFF


def _split_sd(sd):
    src16 = jnp.bitwise_and(sd, MASK16)
    dst16 = lax.shift_right_logical(sd, 16)
    return src16, dst16


def _tc_body(x_ref, w_ref, att8_ref, ht_ref, a8_ref):
    xb = x_ref[...]
    ht = lax.dot_general(w_ref[...], xb, (((0,), (1,)), ((), ())),
                         preferred_element_type=jnp.float32)
    ht_ref[...] = ht
    a8_ref[...] = lax.dot_general(att8_ref[...], ht, (((1,), (0,)), ((), ())),
                                  preferred_element_type=jnp.float32)


def _k1(x, W, att8):
    bn = 1280
    grid = N_PAD // bn
    return pl.pallas_call(
        _tc_body,
        grid=(grid,),
        in_specs=[
            pl.BlockSpec((bn, D_IN), lambda i: (i, 0)),
            pl.BlockSpec((D_IN, D_OUT), lambda i: (0, 0)),
            pl.BlockSpec((8, D_IN), lambda i: (0, 0)),
        ],
        out_specs=[
            pl.BlockSpec((D_OUT, bn), lambda i: (0, i)),
            pl.BlockSpec((8, bn), lambda i: (0, i)),
        ],
        out_shape=[
            jax.ShapeDtypeStruct((D_OUT, N_PAD), jnp.float32),
            jax.ShapeDtypeStruct((8, N_PAD), jnp.float32),
        ],
    )(x, W, att8)


def _k2_body(sd_hbm, a8_hbm, w_hbm, dpart_hbm,
             asrc_v, adst_v, den_v, sd_v, wv_v, red_v, out_v, shared):
    c = lax.axis_index("c")
    s = lax.axis_index("s")
    wid = c * NS + s
    pltpu.sync_copy(a8_hbm.at[0], asrc_v)
    pltpu.sync_copy(a8_hbm.at[1], adst_v)

    @plsc.parallel_loop(0, N_PAD // 16, unroll=4)
    def zero_body(j):
        den_v[pl.ds(j * 16, 16)] = jnp.zeros((16,), jnp.float32)

    ebase = wid * E_PER_W

    def chunk_body(k, _):
        base = ebase + k * CHUNK
        pltpu.sync_copy(sd_hbm.at[pl.ds(base, CHUNK)], sd_v)

        @plsc.parallel_loop(0, CHUNK // 16, unroll=16)
        def grp_body(g):
            src16, dst16 = _split_sd(sd_v[pl.ds(g * 16, 16)])
            al = (plsc.load_gather(asrc_v, [src16])
                  + plsc.load_gather(adst_v, [dst16]))
            al = jnp.where(al >= 0.0, al, al * SLOPE)
            w16 = jnp.exp(al)
            wv_v[pl.ds(g * 16, 16)] = w16
            plsc.addupdate_scatter(den_v, [dst16], w16)
        pltpu.sync_copy(wv_v, w_hbm.at[pl.ds(base, CHUNK)])
        return 0
    lax.fori_loop(0, E_PER_W // CHUNK, chunk_body, 0)

    pltpu.sync_copy(den_v, shared.at[s])
    plsc.subcore_barrier()
    pltpu.sync_copy(shared.at[:, pl.ds(s * SLICE, SLICE)], red_v)

    @plsc.parallel_loop(0, SLICE // 16, unroll=2)
    def red_body(j):
        acc = jnp.zeros((16,), jnp.float32)
        for r in range(NS):
            acc = acc + red_v[r, pl.ds(j * 16, 16)]
        out_v[pl.ds(j * 16, 16)] = acc
    pltpu.sync_copy(out_v, dpart_hbm.at[c, pl.ds(s * SLICE, SLICE)])


def _k2(srcdst, a8):
    mesh = plsc.VectorSubcoreMesh(core_axis_name="c", subcore_axis_name="s")
    f = pl.kernel(
        _k2_body,
        out_type=[
            jax.ShapeDtypeStruct((N_EDGES,), jnp.float32),
            jax.ShapeDtypeStruct((NC, N_PAD), jnp.float32),
        ],
        mesh=mesh,
        compiler_params=pltpu.CompilerParams(needs_layout_passes=False),
        scratch_types=[
            pltpu.VMEM((N_PAD,), jnp.float32),
            pltpu.VMEM((N_PAD,), jnp.float32),
            pltpu.VMEM((N_PAD,), jnp.float32),
            pltpu.VMEM((CHUNK,), jnp.int32),
            pltpu.VMEM((CHUNK,), jnp.float32),
            pltpu.VMEM((NS, SLICE), jnp.float32),
            pltpu.VMEM((SLICE,), jnp.float32),
            pltpu.VMEM_SHARED((NS, N_PAD), jnp.float32),
        ],
    )
    return f(srcdst, a8)


def _k3_body(ht_hbm, sd_hbm, w_hbm, dpart_hbm, bias_hbm, out_hbm,
             hstage_v, hp_v, acc_v, sd_v, wv_v, den_v, den2_v, bias_v, sems):
    c = lax.axis_index("c")
    s = lax.axis_index("s")
    t = c * NS + s
    fbase = t * F_PER_TILE
    nch = N_EDGES // CHUNK3

    def start_chunk(k, slot):
        base = k * CHUNK3
        pltpu.async_copy(sd_hbm.at[pl.ds(base, CHUNK3)], sd_v.at[slot],
                         sems.at[slot])
        pltpu.async_copy(w_hbm.at[pl.ds(base, CHUNK3)], wv_v.at[slot],
                         sems.at[slot])

    def wait_chunk(slot):
        pltpu.make_async_copy(sd_hbm.at[pl.ds(0, CHUNK3)], sd_v.at[slot],
                              sems.at[slot]).wait()
        pltpu.make_async_copy(w_hbm.at[pl.ds(0, CHUNK3)], wv_v.at[slot],
                              sems.at[slot]).wait()

    start_chunk(0, 0)

    for p in range(F_PER_TILE // 2):
        pltpu.sync_copy(ht_hbm.at[pl.ds(fbase + 2 * p, 2)], hstage_v)

        @plsc.parallel_loop(0, N_PAD // 16, unroll=4)
        def pack_body(j, p=p):
            a = hstage_v[0, pl.ds(j * 16, 16)]
            b = hstage_v[1, pl.ds(j * 16, 16)]
            packed = plsc.pack(a, b, format=plsc.PackFormat.INTERLEAVED)
            hp_v[p, pl.ds(j * 16, 16)] = plsc.bitcast(packed, jnp.int32)

    @plsc.parallel_loop(0, N_PAD // 16, unroll=4)
    def zero_body(j):
        for f in range(F_PER_TILE):
            acc_v[f, pl.ds(j * 16, 16)] = jnp.zeros((16,), jnp.float32)

    def chunk_body(k, _):
        slot = lax.rem(k, 2)

        @pl.when(k + 1 < nch)
        def _():
            start_chunk(k + 1, 1 - slot)
        wait_chunk(slot)

        @plsc.parallel_loop(0, CHUNK3 // 16, unroll=8)
        def grp_body(g):
            src16, dst16 = _split_sd(sd_v[slot, pl.ds(g * 16, 16)])
            w16 = wv_v[slot, pl.ds(g * 16, 16)]
            for p in range(F_PER_TILE // 2):
                pidx = jnp.full((16,), p, jnp.int32)
                words = plsc.load_gather(hp_v, [pidx, src16])
                rows = plsc.unpack(plsc.bitcast(words, jnp.bfloat16),
                                   format=plsc.PackFormat.INTERLEAVED)
                for q in range(2):
                    fidx = jnp.full((16,), 2 * p + q, jnp.int32)
                    plsc.addupdate_scatter(acc_v, [fidx, dst16],
                                           rows[q] * w16)
        return 0
    lax.fori_loop(0, nch, chunk_body, 0)

    pltpu.sync_copy(dpart_hbm.at[0], den_v)
    pltpu.sync_copy(dpart_hbm.at[1], den2_v)
    pltpu.sync_copy(bias_hbm, bias_v)

    bfs = [plsc.load_gather(bias_v, [jnp.full((16,), 1, jnp.int32) * (fbase + f)])
           for f in range(F_PER_TILE)]

    @plsc.parallel_loop(0, N_PAD // 16, unroll=2)
    def norm_body(j):
        dsum = den_v[pl.ds(j * 16, 16)] + den2_v[pl.ds(j * 16, 16)] + 1e-16
        rden = 1.0 / dsum
        for f in range(F_PER_TILE):
            v = acc_v[f, pl.ds(j * 16, 16)] * rden + bfs[f]
            v = jnp.where(v > 0.0, v, jnp.exp(v) - 1.0)
            acc_v[f, pl.ds(j * 16, 16)] = v
    pltpu.sync_copy(acc_v, out_hbm.at[pl.ds(fbase, F_PER_TILE)])


def _k3(ht, srcdst, w_e, dparts, bias):
    mesh = plsc.VectorSubcoreMesh(core_axis_name="c", subcore_axis_name="s")
    f = pl.kernel(
        _k3_body,
        out_type=jax.ShapeDtypeStruct((D_OUT, N_PAD), jnp.float32),
        mesh=mesh,
        compiler_params=pltpu.CompilerParams(needs_layout_passes=False),
        scratch_types=[
            pltpu.VMEM((2, N_PAD), jnp.float32),
            pltpu.VMEM((F_PER_TILE // 2, N_PAD), jnp.int32),
            pltpu.VMEM((F_PER_TILE, N_PAD), jnp.float32),
            pltpu.VMEM((2, CHUNK3), jnp.int32),
            pltpu.VMEM((2, CHUNK3), jnp.float32),
            pltpu.VMEM((N_PAD,), jnp.float32),
            pltpu.VMEM((N_PAD,), jnp.float32),
            pltpu.VMEM((D_OUT,), jnp.float32),
            pltpu.SemaphoreType.DMA((2,)),
        ],
    )
    return f(ht, srcdst, w_e, dparts, bias)


def kernel(x, edge_index, W, att_src, att_dst, bias):
    att8 = jnp.concatenate(
        [att_src, att_dst, jnp.zeros((6, D_IN), jnp.float32)], axis=0)
    xp = jnp.pad(x, ((0, N_PAD - N_NODES), (0, 0)))
    srcdst = jnp.bitwise_or(edge_index[0],
                            lax.shift_left(edge_index[1], 16))
    ht, a8 = _k1(xp, W, att8)
    w_e, dparts = _k2(srcdst, a8)
    out_t = _k3(ht, srcdst, w_e, dparts, bias)
    return out_t[:, :N_NODES].T
